# Initial kernel scaffold; baseline (speedup 1.0000x reference)
import jax
import jax.numpy as jnp
from jax.experimental import pallas as pl


def kernel(z_e, emb_w):
    raise NotImplementedError("write your pallas kernel here")


# speedup vs baseline: 2.0727x; 2.0727x over previous
import functools

import jax
import jax.numpy as jnp
from jax.experimental import pallas as pl
from jax.experimental.pallas import tpu as pltpu

_NUM_EMB = 1024
_COMMIT = 0.25


def _vq_body(z_ref, ew_ref, ewt_ref, zq_ref, idx_ref, loss_ref, perp_ref,
             counts, sq_acc, *, nsteps, total_elems, total_rows):
    i = pl.program_id(0)
    zt = z_ref[0]
    ew = ew_ref[...]
    ewt = ewt_ref[...]

    K = ew.shape[0]
    R = zt.shape[1]

    zwT = jnp.dot(ew, zt, preferred_element_type=jnp.float32)
    z2 = jnp.sum(zt * zt, axis=0, keepdims=True)
    e2 = jnp.sum(ew * ew, axis=1, keepdims=True)
    dT = (z2 - 2.0 * zwT) + e2

    dmin = jnp.min(dT, axis=0, keepdims=True)
    iota0 = jax.lax.broadcasted_iota(jnp.int32, (K, R), 0)
    cand = jnp.where(dT == dmin, iota0, K)
    idx = jnp.min(cand, axis=0, keepdims=True)

    ohT = (iota0 == idx).astype(jnp.float32)
    zqT = jnp.dot(ewt, ohT, preferred_element_type=jnp.float32)

    zq_ref[0] = zt + (zqT - zt)
    idx_ref[0] = idx

    sq = jnp.sum((zt - zqT) ** 2)
    cnt = jnp.sum(ohT, axis=1, keepdims=True)

    @pl.when(i == 0)
    def _init():
        counts[...] = cnt
        sq_acc[0, 0] = sq

    @pl.when(i != 0)
    def _acc():
        counts[...] = counts[...] + cnt
        sq_acc[0, 0] = sq_acc[0, 0] + sq

    @pl.when(i == nsteps - 1)
    def _final():
        loss = (1.0 + _COMMIT) * sq_acc[0, 0] / total_elems
        loss_ref[...] = jnp.full((1, 1), loss, jnp.float32)
        p = counts[...] * (1.0 / total_rows)
        ent = jnp.sum(p * jnp.log(jnp.maximum(p, 1e-10)), keepdims=True)
        perp_ref[...] = jnp.exp(-ent)


@jax.jit
def _vq(z3, ew, ewt):
    b, c, r = z3.shape
    k = ew.shape[0]
    body = functools.partial(
        _vq_body, nsteps=b, total_elems=float(b * c * r), total_rows=float(b * r))
    out_shape = (
        jax.ShapeDtypeStruct((b, c, r), jnp.float32),
        jax.ShapeDtypeStruct((b, 1, r), jnp.int32),
        jax.ShapeDtypeStruct((1, 1), jnp.float32),
        jax.ShapeDtypeStruct((1, 1), jnp.float32),
    )
    grid = (b,)
    zq, idx, loss, perp = pl.pallas_call(
        body,
        grid=grid,
        in_specs=[
            pl.BlockSpec((1, c, r), lambda i: (i, 0, 0)),
            pl.BlockSpec((k, c), lambda i: (0, 0)),
            pl.BlockSpec((c, k), lambda i: (0, 0)),
        ],
        out_specs=(
            pl.BlockSpec((1, c, r), lambda i: (i, 0, 0)),
            pl.BlockSpec((1, 1, r), lambda i: (i, 0, 0)),
            pl.BlockSpec((1, 1), lambda i: (0, 0)),
            pl.BlockSpec((1, 1), lambda i: (0, 0)),
        ),
        out_shape=out_shape,
        scratch_shapes=[
            pltpu.VMEM((k, 1), jnp.float32),
            pltpu.SMEM((1, 1), jnp.float32),
        ],
    )(z3, ew, ewt)
    return zq, idx, loss, perp


def kernel(z_e, emb_w):
    b, c, h, w = z_e.shape
    z3 = z_e.astype(jnp.float32).reshape(b, c, h * w)
    ew = emb_w.astype(jnp.float32)
    zq, idx, loss, perp = _vq(z3, ew, ew.T)
    z_q_st = zq.reshape(b, c, h, w)
    indices = idx.reshape(b, h, w)
    return (z_q_st, loss.reshape(()), perp.reshape(()), indices)
